# Initial kernel scaffold; baseline (speedup 1.0000x reference)
import jax
import jax.numpy as jnp
from jax.experimental import pallas as pl


def kernel(x, geom):
    raise NotImplementedError("write your pallas kernel here")


# speedup vs baseline: 1.4614x; 1.4614x over previous
import functools

import jax
import jax.numpy as jnp
from jax import lax
from jax.experimental import pallas as pl
from jax.experimental.pallas import tpu as pltpu
from jax.experimental.pallas import tpu_sc as plsc

B, N, D, H, W, C = 4, 6, 41, 8, 22, 64
NP = B * N * D * H * W
NPB = NP // B
NX = 200
GR = NX * NX
TRASH = 128
GRT = GR + TRASH
BLK = 128
NBLK = NP // BLK
NS = 16
NCORE = 2
CH = C // NCORE
MAXPB = -(-NBLK // NS)
PASS_LO = [(b * NPB) // BLK for b in range(B)]
PASS_HI = [((b + 1) * NPB - 1) // BLK for b in range(B)]
MAXIT = 22
WROWS = 2496
NCHUNK = 19


@functools.partial(
    pl.kernel,
    mesh=plsc.VectorSubcoreMesh(core_axis_name="c", subcore_axis_name="s"),
    out_type=jax.ShapeDtypeStruct((NCORE, B, GR, CH), jnp.float32),
    compiler_params=pltpu.CompilerParams(use_tc_tiling_on_sc=False),
    scratch_types=[
        pltpu.VMEM((MAXPB * BLK,), jnp.int32),
        pltpu.VMEM((3, BLK), jnp.float32),
        pltpu.VMEM((BLK, C), jnp.float32),
        pltpu.VMEM((BLK, CH), jnp.float32),
        pltpu.VMEM((BLK,), jnp.int32),
        pltpu.VMEM_SHARED((GRT, CH), jnp.float32),
    ],
)
def _bev_scatter(x_hbm, geomt_hbm, out_hbm,
                 prank_v, gbuf, xbuf, xc, idxbuf, grid):
    c = lax.axis_index("c")
    s = lax.axis_index("s")
    lanes = jnp.arange(16, dtype=jnp.int32)
    zvec = jnp.zeros((16,), jnp.float32)

    def voxelize(i, carry):
        bi = s + i * NS

        @pl.when(bi < NBLK)
        def _():
            pltpu.sync_copy(geomt_hbm.at[:, pl.ds(bi * BLK, BLK)], gbuf)
            for j in range(BLK // 16):
                gx = gbuf[0, pl.ds(j * 16, 16)]
                gy = gbuf[1, pl.ds(j * 16, 16)]
                gz = gbuf[2, pl.ds(j * 16, 16)]
                ix = ((gx - jnp.float32(-50.0)) / jnp.float32(0.5)
                      ).astype(jnp.int32)
                iy = ((gy - jnp.float32(-50.0)) / jnp.float32(0.5)
                      ).astype(jnp.int32)
                iz = ((gz - jnp.float32(-10.0)) / jnp.float32(20.0)
                      ).astype(jnp.int32)
                kept = ((ix >= 0) & (ix < NX) & (iy >= 0) & (iy < NX)
                        & (iz == 0))
                pr = jnp.where(kept, ix * NX + iy, jnp.int32(-1))
                prank_v[pl.ds(i * BLK + j * 16, 16)] = pr
        return carry

    lax.fori_loop(0, MAXPB, voxelize, 0)

    for b in range(B):
        lo_blk, hi_blk = PASS_LO[b], PASS_HI[b]
        lmod = lo_blk % NS
        off = jnp.bitwise_and(s - lmod + NS, NS - 1)
        bi0 = lo_blk + off
        slot0 = lax.shift_right_logical(bi0 - s, 4)
        lob = jnp.int32(b * NPB)
        hib = jnp.int32((b + 1) * NPB)

        def zero_xc(r, carry):
            xc[r, pl.ds(0, 16)] = zvec
            xc[r, pl.ds(16, 16)] = zvec
            return carry

        lax.fori_loop(0, BLK, zero_xc, 0)

        def zero_grid(k, carry):
            pltpu.sync_copy(xc, grid.at[pl.ds(s * WROWS + k * BLK, BLK), :])
            return carry

        lax.fori_loop(0, NCHUNK, zero_grid, 0)
        pltpu.sync_copy(xc.at[pl.ds(0, 64), :],
                        grid.at[pl.ds(s * WROWS + NCHUNK * BLK, 64), :])

        @pl.when(s == 0)
        def _():
            pltpu.sync_copy(xc, grid.at[pl.ds(NS * WROWS, BLK), :])
            pltpu.sync_copy(xc.at[pl.ds(0, 64), :],
                            grid.at[pl.ds(NS * WROWS + BLK, 64), :])
        plsc.subcore_barrier()

        def scatter(i, carry, bi0=bi0, slot0=slot0, hi_blk=hi_blk,
                    lob=lob, hib=hib):
            bi = bi0 + i * NS

            @pl.when(bi <= hi_blk)
            def _():
                pltpu.sync_copy(x_hbm.at[pl.ds(bi * BLK, BLK), :], xbuf)
                base = (slot0 + i) * BLK
                for j in range(BLK // 16):
                    pr = prank_v[pl.ds(base + j * 16, 16)]
                    gi = bi * BLK + j * 16 + lanes
                    okv = (pr >= 0) & (gi >= lob) & (gi < hib)
                    idx = jnp.where(okv, pr, jnp.int32(GR + j * 16) + lanes)
                    idxbuf[pl.ds(j * 16, 16)] = idx
                c32 = c * CH
                for p in range(BLK):
                    xc[p, pl.ds(0, 16)] = xbuf[p, pl.ds(c32, 16)]
                    xc[p, pl.ds(16, 16)] = xbuf[p, pl.ds(c32 + 16, 16)]
                pltpu.sync_copy(xc, grid.at[idxbuf], add=True)
            return carry

        lax.fori_loop(0, MAXIT, scatter, 0)
        plsc.subcore_barrier()

        def wb_grid(k, carry, b=b):
            r0 = s * WROWS + k * BLK
            pltpu.sync_copy(grid.at[pl.ds(r0, BLK), :], xc)
            pltpu.sync_copy(xc, out_hbm.at[c, b, pl.ds(r0, BLK), :])
            return carry

        lax.fori_loop(0, NCHUNK, wb_grid, 0)
        r1 = s * WROWS + NCHUNK * BLK
        pltpu.sync_copy(grid.at[pl.ds(r1, 64), :], xc.at[pl.ds(0, 64), :])
        pltpu.sync_copy(xc.at[pl.ds(0, 64), :],
                        out_hbm.at[c, b, pl.ds(r1, 64), :])

        @pl.when(s == 0)
        def _():
            pltpu.sync_copy(grid.at[pl.ds(NS * WROWS, 64), :],
                            xc.at[pl.ds(0, 64), :])
            pltpu.sync_copy(xc.at[pl.ds(0, 64), :],
                            out_hbm.at[c, b, pl.ds(NS * WROWS, 64), :])
        plsc.subcore_barrier()


@jax.jit
def kernel(x, geom):
    x2 = x.reshape(NP, C)
    geomt = geom.reshape(NP, 3).T
    out = _bev_scatter(x2, geomt)
    out = out.reshape(NCORE, B, NX, NX, CH)
    return out.transpose(1, 0, 4, 2, 3).reshape(B, C, NX, NX)


# speedup vs baseline: 1.4749x; 1.0092x over previous
import functools

import jax
import jax.numpy as jnp
from jax import lax
from jax.experimental import pallas as pl
from jax.experimental.pallas import tpu as pltpu
from jax.experimental.pallas import tpu_sc as plsc

B, N, D, H, W, C = 4, 6, 41, 8, 22, 64
NP = B * N * D * H * W
NPB = NP // B
NX = 200
GR = NX * NX
TRASH = 128
GRT = GR + TRASH
BLK = 128
NBLK = NP // BLK
NS = 16
NCORE = 2
CH = C // NCORE
MAXPB = -(-NBLK // NS)
PASS_LO = [(b * NPB) // BLK for b in range(B)]
PASS_HI = [((b + 1) * NPB - 1) // BLK for b in range(B)]
MAXIT = 22
WROWS = 2496
NCHUNK = 19


@functools.partial(
    pl.kernel,
    mesh=plsc.VectorSubcoreMesh(core_axis_name="c", subcore_axis_name="s"),
    out_type=jax.ShapeDtypeStruct((NCORE, B, GR, CH), jnp.float32),
    compiler_params=pltpu.CompilerParams(use_tc_tiling_on_sc=False),
    scratch_types=[
        pltpu.VMEM((MAXPB * BLK,), jnp.int32),
        pltpu.VMEM((3, BLK), jnp.float32),
        pltpu.VMEM((BLK, C), jnp.float32),
        pltpu.VMEM((BLK, CH), jnp.float32),
        pltpu.VMEM((BLK,), jnp.int32),
        pltpu.VMEM((624, CH), jnp.float32),
        pltpu.VMEM_SHARED((GRT, CH), jnp.float32),
    ],
)
def _bev_scatter(x_hbm, geomt_hbm, out_hbm,
                 prank_v, gbuf, xbuf, xc, idxbuf, wc, grid):
    c = lax.axis_index("c")
    s = lax.axis_index("s")
    lanes = jnp.arange(16, dtype=jnp.int32)
    zvec = jnp.zeros((16,), jnp.float32)

    def voxelize(i, carry):
        bi = s + i * NS

        @pl.when(bi < NBLK)
        def _():
            pltpu.sync_copy(geomt_hbm.at[:, pl.ds(bi * BLK, BLK)], gbuf)
            for j in range(BLK // 16):
                gx = gbuf[0, pl.ds(j * 16, 16)]
                gy = gbuf[1, pl.ds(j * 16, 16)]
                gz = gbuf[2, pl.ds(j * 16, 16)]
                ix = ((gx - jnp.float32(-50.0)) / jnp.float32(0.5)
                      ).astype(jnp.int32)
                iy = ((gy - jnp.float32(-50.0)) / jnp.float32(0.5)
                      ).astype(jnp.int32)
                iz = ((gz - jnp.float32(-10.0)) / jnp.float32(20.0)
                      ).astype(jnp.int32)
                kept = ((ix >= 0) & (ix < NX) & (iy >= 0) & (iy < NX)
                        & (iz == 0))
                pr = jnp.where(kept, ix * NX + iy, jnp.int32(-1))
                prank_v[pl.ds(i * BLK + j * 16, 16)] = pr
        return carry

    lax.fori_loop(0, MAXPB, voxelize, 0)

    for b in range(B):
        lo_blk, hi_blk = PASS_LO[b], PASS_HI[b]
        lmod = lo_blk % NS
        off = jnp.bitwise_and(s - lmod + NS, NS - 1)
        bi0 = lo_blk + off
        slot0 = lax.shift_right_logical(bi0 - s, 4)
        lob = jnp.int32(b * NPB)
        hib = jnp.int32((b + 1) * NPB)

        def zero_wc(r, carry):
            wc[r, pl.ds(0, 16)] = zvec
            wc[r, pl.ds(16, 16)] = zvec
            return carry

        lax.fori_loop(0, 624, zero_wc, 0)

        def zero_grid(k, carry):
            pltpu.sync_copy(wc, grid.at[pl.ds(s * WROWS + k * 624, 624), :])
            return carry

        lax.fori_loop(0, 4, zero_grid, 0)

        @pl.when(s == 0)
        def _():
            pltpu.sync_copy(wc.at[pl.ds(0, 192), :],
                            grid.at[pl.ds(NS * WROWS, 192), :])
        plsc.subcore_barrier()

        def scatter(i, carry, bi0=bi0, slot0=slot0, hi_blk=hi_blk,
                    lob=lob, hib=hib):
            bi = bi0 + i * NS

            @pl.when(bi <= hi_blk)
            def _():
                pltpu.sync_copy(x_hbm.at[pl.ds(bi * BLK, BLK), :], xbuf)
                base = (slot0 + i) * BLK
                for j in range(BLK // 16):
                    pr = prank_v[pl.ds(base + j * 16, 16)]
                    gi = bi * BLK + j * 16 + lanes
                    okv = (pr >= 0) & (gi >= lob) & (gi < hib)
                    idx = jnp.where(okv, pr, jnp.int32(GR + j * 16) + lanes)
                    idxbuf[pl.ds(j * 16, 16)] = idx
                c32 = c * CH
                for p in range(BLK):
                    xc[p, pl.ds(0, 16)] = xbuf[p, pl.ds(c32, 16)]
                    xc[p, pl.ds(16, 16)] = xbuf[p, pl.ds(c32 + 16, 16)]
                pltpu.sync_copy(xc, grid.at[idxbuf], add=True)
            return carry

        lax.fori_loop(0, MAXIT, scatter, 0)
        plsc.subcore_barrier()

        def wb_grid(k, carry, b=b):
            r0 = s * WROWS + k * 624
            pltpu.sync_copy(grid.at[pl.ds(r0, 624), :], wc)
            pltpu.sync_copy(wc, out_hbm.at[c, b, pl.ds(r0, 624), :])
            return carry

        lax.fori_loop(0, 4, wb_grid, 0)

        @pl.when(s == 0)
        def _():
            pltpu.sync_copy(grid.at[pl.ds(NS * WROWS, 64), :],
                            wc.at[pl.ds(0, 64), :])
            pltpu.sync_copy(wc.at[pl.ds(0, 64), :],
                            out_hbm.at[c, b, pl.ds(NS * WROWS, 64), :])
        plsc.subcore_barrier()


@jax.jit
def kernel(x, geom):
    x2 = x.reshape(NP, C)
    geomt = geom.reshape(NP, 3).T
    out = _bev_scatter(x2, geomt)
    out = out.reshape(NCORE, B, NX, NX, CH)
    return out.transpose(1, 0, 4, 2, 3).reshape(B, C, NX, NX)


# speedup vs baseline: 1.6785x; 1.1380x over previous
import functools

import jax
import jax.numpy as jnp
from jax import lax
from jax.experimental import pallas as pl
from jax.experimental.pallas import tpu as pltpu
from jax.experimental.pallas import tpu_sc as plsc

B, N, D, H, W, C = 4, 6, 41, 8, 22, 64
NP = B * N * D * H * W
NPB = NP // B
NX = 200
GR = NX * NX
TRASH = 128
GRT = GR + TRASH
BLK = 128
NBLK = NP // BLK
NS = 16
NCORE = 2
CH = C // NCORE
MAXPB = -(-NBLK // NS)
PASS_LO = [(b * NPB) // BLK for b in range(B)]
PASS_HI = [((b + 1) * NPB - 1) // BLK for b in range(B)]
MAXIT = 22
WROWS = 2496
NCHUNK = 19


@functools.partial(
    pl.kernel,
    mesh=plsc.VectorSubcoreMesh(core_axis_name="c", subcore_axis_name="s"),
    out_type=jax.ShapeDtypeStruct((NCORE, B, GR, CH), jnp.float32),
    compiler_params=pltpu.CompilerParams(use_tc_tiling_on_sc=False),
    scratch_types=[
        pltpu.VMEM((MAXPB * BLK,), jnp.int32),
        pltpu.VMEM((3, BLK), jnp.float32),
        pltpu.VMEM((BLK, C), jnp.float32),
        pltpu.VMEM((BLK, C), jnp.float32),
        pltpu.VMEM((BLK, CH), jnp.float32),
        pltpu.VMEM((BLK,), jnp.int32),
        pltpu.VMEM((512, CH), jnp.float32),
        pltpu.VMEM_SHARED((GRT, CH), jnp.float32),
        pltpu.SemaphoreType.DMA,
        pltpu.SemaphoreType.DMA,
    ],
)
def _bev_scatter(x_hbm, geomt_hbm, out_hbm,
                 prank_v, gbuf, xbuf, xbuf2, xc, idxbuf, wc, grid, sem_a, sem_b):
    c = lax.axis_index("c")
    s = lax.axis_index("s")
    lanes = jnp.arange(16, dtype=jnp.int32)
    zvec = jnp.zeros((16,), jnp.float32)

    def voxelize(i, carry):
        bi = s + i * NS

        @pl.when(bi < NBLK)
        def _():
            pltpu.sync_copy(geomt_hbm.at[:, pl.ds(bi * BLK, BLK)], gbuf)
            for j in range(BLK // 16):
                gx = gbuf[0, pl.ds(j * 16, 16)]
                gy = gbuf[1, pl.ds(j * 16, 16)]
                gz = gbuf[2, pl.ds(j * 16, 16)]
                ix = ((gx - jnp.float32(-50.0)) / jnp.float32(0.5)
                      ).astype(jnp.int32)
                iy = ((gy - jnp.float32(-50.0)) / jnp.float32(0.5)
                      ).astype(jnp.int32)
                iz = ((gz - jnp.float32(-10.0)) / jnp.float32(20.0)
                      ).astype(jnp.int32)
                kept = ((ix >= 0) & (ix < NX) & (iy >= 0) & (iy < NX)
                        & (iz == 0))
                pr = jnp.where(kept, ix * NX + iy, jnp.int32(-1))
                prank_v[pl.ds(i * BLK + j * 16, 16)] = pr
        return carry

    lax.fori_loop(0, MAXPB, voxelize, 0)

    for b in range(B):
        lo_blk, hi_blk = PASS_LO[b], PASS_HI[b]
        lmod = lo_blk % NS
        off = jnp.bitwise_and(s - lmod + NS, NS - 1)
        bi0 = lo_blk + off
        slot0 = lax.shift_right_logical(bi0 - s, 4)
        lob = jnp.int32(b * NPB)
        hib = jnp.int32((b + 1) * NPB)

        def zero_wc(r, carry):
            wc[r, pl.ds(0, 16)] = zvec
            wc[r, pl.ds(16, 16)] = zvec
            return carry

        lax.fori_loop(0, 512, zero_wc, 0)

        def zero_grid(k, carry):
            pltpu.sync_copy(wc, grid.at[pl.ds(s * WROWS + k * 512, 512), :])
            return carry

        lax.fori_loop(0, 4, zero_grid, 0)
        pltpu.sync_copy(wc.at[pl.ds(0, 448), :],
                        grid.at[pl.ds(s * WROWS + 2048, 448), :])

        @pl.when(s == 0)
        def _():
            pltpu.sync_copy(wc.at[pl.ds(0, 192), :],
                            grid.at[pl.ds(NS * WROWS, 192), :])
        plsc.subcore_barrier()

        def load_x(bi, buf, sem):
            return pltpu.make_async_copy(
                x_hbm.at[pl.ds(bi * BLK, BLK), :], buf, sem)

        def do_block(bi, i, buf):
            base = (slot0 + i) * BLK
            for j in range(BLK // 16):
                pr = prank_v[pl.ds(base + j * 16, 16)]
                gi = bi * BLK + j * 16 + lanes
                okv = (pr >= 0) & (gi >= lob) & (gi < hib)
                idx = jnp.where(okv, pr, jnp.int32(GR + j * 16) + lanes)
                idxbuf[pl.ds(j * 16, 16)] = idx
            c32 = c * CH
            for p in range(BLK):
                xc[p, pl.ds(0, 16)] = buf[p, pl.ds(c32, 16)]
                xc[p, pl.ds(16, 16)] = buf[p, pl.ds(c32 + 16, 16)]
            pltpu.sync_copy(xc, grid.at[idxbuf], add=True)

        @pl.when(bi0 <= hi_blk)
        def _():
            load_x(bi0, xbuf, sem_a).start()

        def scatter(i, carry, bi0=bi0, slot0=slot0, hi_blk=hi_blk,
                    lob=lob, hib=hib):
            bi_a = bi0 + (2 * i) * NS
            bi_b = bi_a + NS
            bi_c = bi_b + NS

            @pl.when(bi_a <= hi_blk)
            def _():
                load_x(bi_a, xbuf, sem_a).wait()

                @pl.when(bi_b <= hi_blk)
                def _():
                    load_x(bi_b, xbuf2, sem_b).start()
                do_block(bi_a, 2 * i, xbuf)

            @pl.when(bi_b <= hi_blk)
            def _():
                load_x(bi_b, xbuf2, sem_b).wait()

                @pl.when(bi_c <= hi_blk)
                def _():
                    load_x(bi_c, xbuf, sem_a).start()
                do_block(bi_b, 2 * i + 1, xbuf2)
            return carry

        lax.fori_loop(0, MAXIT // 2, scatter, 0)
        plsc.subcore_barrier()

        def wb_grid(k, carry, b=b):
            r0 = s * WROWS + k * 512
            pltpu.sync_copy(grid.at[pl.ds(r0, 512), :], wc)
            pltpu.sync_copy(wc, out_hbm.at[c, b, pl.ds(r0, 512), :])
            return carry

        lax.fori_loop(0, 4, wb_grid, 0)
        r2 = s * WROWS + 2048
        pltpu.sync_copy(grid.at[pl.ds(r2, 448), :], wc.at[pl.ds(0, 448), :])
        pltpu.sync_copy(wc.at[pl.ds(0, 448), :],
                        out_hbm.at[c, b, pl.ds(r2, 448), :])

        @pl.when(s == 0)
        def _():
            pltpu.sync_copy(grid.at[pl.ds(NS * WROWS, 64), :],
                            wc.at[pl.ds(0, 64), :])
            pltpu.sync_copy(wc.at[pl.ds(0, 64), :],
                            out_hbm.at[c, b, pl.ds(NS * WROWS, 64), :])
        plsc.subcore_barrier()


@jax.jit
def kernel(x, geom):
    x2 = x.reshape(NP, C)
    geomt = geom.reshape(NP, 3).T
    out = _bev_scatter(x2, geomt)
    out = out.reshape(NCORE, B, NX, NX, CH)
    return out.transpose(1, 0, 4, 2, 3).reshape(B, C, NX, NX)


# speedup vs baseline: 1.7235x; 1.0268x over previous
import functools

import jax
import jax.numpy as jnp
from jax import lax
from jax.experimental import pallas as pl
from jax.experimental.pallas import tpu as pltpu
from jax.experimental.pallas import tpu_sc as plsc

B, N, D, H, W, C = 4, 6, 41, 8, 22, 64
NP = B * N * D * H * W
NPB = NP // B
NX = 200
GR = NX * NX
TRASH = 128
GRT = GR + TRASH
BLK = 128
NBLK = NP // BLK
NS = 16
NCORE = 2
CH = C // NCORE
MAXPB = -(-NBLK // NS)
PASS_LO = [(b * NPB) // BLK for b in range(B)]
PASS_HI = [((b + 1) * NPB - 1) // BLK for b in range(B)]
MAXIT = 22
WROWS = 2496
NCHUNK = 19


@functools.partial(
    pl.kernel,
    mesh=plsc.VectorSubcoreMesh(core_axis_name="c", subcore_axis_name="s"),
    out_type=jax.ShapeDtypeStruct((NCORE, B, GR, CH), jnp.float32),
    compiler_params=pltpu.CompilerParams(use_tc_tiling_on_sc=False),
    scratch_types=[
        pltpu.VMEM((MAXPB * BLK,), jnp.int32),
        pltpu.VMEM((3, BLK), jnp.float32),
        pltpu.VMEM((BLK, C), jnp.float32),
        pltpu.VMEM((BLK, C), jnp.float32),
        pltpu.VMEM((BLK, CH), jnp.float32),
        pltpu.VMEM((BLK, CH), jnp.float32),
        pltpu.VMEM((BLK,), jnp.int32),
        pltpu.VMEM((BLK,), jnp.int32),
        pltpu.VMEM((416, CH), jnp.float32),
        pltpu.VMEM_SHARED((GRT, CH), jnp.float32),
        pltpu.SemaphoreType.DMA,
        pltpu.SemaphoreType.DMA,
        pltpu.SemaphoreType.DMA,
        pltpu.SemaphoreType.DMA,
    ],
)
def _bev_scatter(x_hbm, geomt_hbm, out_hbm,
                 prank_v, gbuf, xbuf, xbuf2, xc, xc2, idxbuf, idxbuf2, wc, grid,
                 sem_a, sem_b, sem_sa, sem_sb):
    c = lax.axis_index("c")
    s = lax.axis_index("s")
    lanes = jnp.arange(16, dtype=jnp.int32)
    zvec = jnp.zeros((16,), jnp.float32)

    def voxelize(i, carry):
        bi = s + i * NS

        @pl.when(bi < NBLK)
        def _():
            pltpu.sync_copy(geomt_hbm.at[:, pl.ds(bi * BLK, BLK)], gbuf)
            for j in range(BLK // 16):
                gx = gbuf[0, pl.ds(j * 16, 16)]
                gy = gbuf[1, pl.ds(j * 16, 16)]
                gz = gbuf[2, pl.ds(j * 16, 16)]
                ix = ((gx - jnp.float32(-50.0)) / jnp.float32(0.5)
                      ).astype(jnp.int32)
                iy = ((gy - jnp.float32(-50.0)) / jnp.float32(0.5)
                      ).astype(jnp.int32)
                iz = ((gz - jnp.float32(-10.0)) / jnp.float32(20.0)
                      ).astype(jnp.int32)
                kept = ((ix >= 0) & (ix < NX) & (iy >= 0) & (iy < NX)
                        & (iz == 0))
                pr = jnp.where(kept, ix * NX + iy, jnp.int32(-1))
                prank_v[pl.ds(i * BLK + j * 16, 16)] = pr
        return carry

    lax.fori_loop(0, MAXPB, voxelize, 0)

    for b in range(B):
        lo_blk, hi_blk = PASS_LO[b], PASS_HI[b]
        lmod = lo_blk % NS
        off = jnp.bitwise_and(s - lmod + NS, NS - 1)
        bi0 = lo_blk + off
        slot0 = lax.shift_right_logical(bi0 - s, 4)
        lob = jnp.int32(b * NPB)
        hib = jnp.int32((b + 1) * NPB)

        def zero_wc(r, carry):
            wc[r, pl.ds(0, 16)] = zvec
            wc[r, pl.ds(16, 16)] = zvec
            return carry

        lax.fori_loop(0, 416, zero_wc, 0)

        def zero_grid(k, carry):
            pltpu.sync_copy(wc, grid.at[pl.ds(s * WROWS + k * 416, 416), :])
            return carry

        lax.fori_loop(0, 6, zero_grid, 0)

        @pl.when(s == 0)
        def _():
            pltpu.sync_copy(wc.at[pl.ds(0, 192), :],
                            grid.at[pl.ds(NS * WROWS, 192), :])
        plsc.subcore_barrier()

        def load_x(bi, buf, sem):
            return pltpu.make_async_copy(
                x_hbm.at[pl.ds(bi * BLK, BLK), :], buf, sem)

        def do_block(bi, i, buf, xcb, idxb, sem_s):
            base = (slot0 + i) * BLK
            for j in range(BLK // 16):
                pr = prank_v[pl.ds(base + j * 16, 16)]
                gi = bi * BLK + j * 16 + lanes
                okv = (pr >= 0) & (gi >= lob) & (gi < hib)
                idx = jnp.where(okv, pr, jnp.int32(GR + j * 16) + lanes)
                idxb[pl.ds(j * 16, 16)] = idx
            c32 = c * CH
            for p in range(BLK):
                xcb[p, pl.ds(0, 16)] = buf[p, pl.ds(c32, 16)]
                xcb[p, pl.ds(16, 16)] = buf[p, pl.ds(c32 + 16, 16)]
            pltpu.make_async_copy(xcb, grid.at[idxb], sem_s).start(add=True)

        @pl.when(bi0 <= hi_blk)
        def _():
            load_x(bi0, xbuf, sem_a).start()

        def scatter(i, carry, bi0=bi0, slot0=slot0, hi_blk=hi_blk,
                    lob=lob, hib=hib):
            bi_a = bi0 + (2 * i) * NS
            bi_b = bi_a + NS
            bi_c = bi_b + NS

            @pl.when(bi_a <= hi_blk)
            def _():
                load_x(bi_a, xbuf, sem_a).wait()

                @pl.when(bi_b <= hi_blk)
                def _():
                    load_x(bi_b, xbuf2, sem_b).start()

                @pl.when(bi_a - 2 * NS >= bi0)
                def _():
                    pltpu.make_async_copy(xc, grid.at[idxbuf],
                                          sem_sa).wait()
                do_block(bi_a, 2 * i, xbuf, xc, idxbuf, sem_sa)

            @pl.when(bi_b <= hi_blk)
            def _():
                load_x(bi_b, xbuf2, sem_b).wait()

                @pl.when(bi_c <= hi_blk)
                def _():
                    load_x(bi_c, xbuf, sem_a).start()

                @pl.when(bi_b - 2 * NS >= bi0)
                def _():
                    pltpu.make_async_copy(xc2, grid.at[idxbuf2],
                                          sem_sb).wait()
                do_block(bi_b, 2 * i + 1, xbuf2, xc2, idxbuf2, sem_sb)
            return carry

        lax.fori_loop(0, MAXIT // 2, scatter, 0)

        @pl.when(bi0 <= hi_blk)
        def _():
            pltpu.make_async_copy(xc, grid.at[idxbuf], sem_sa).wait()

        @pl.when(bi0 + NS <= hi_blk)
        def _():
            pltpu.make_async_copy(xc2, grid.at[idxbuf2], sem_sb).wait()
        plsc.subcore_barrier()

        def wb_grid(k, carry, b=b):
            r0 = s * WROWS + k * 416
            pltpu.sync_copy(grid.at[pl.ds(r0, 416), :], wc)
            pltpu.sync_copy(wc, out_hbm.at[c, b, pl.ds(r0, 416), :])
            return carry

        lax.fori_loop(0, 6, wb_grid, 0)

        @pl.when(s == 0)
        def _():
            pltpu.sync_copy(grid.at[pl.ds(NS * WROWS, 64), :],
                            wc.at[pl.ds(0, 64), :])
            pltpu.sync_copy(wc.at[pl.ds(0, 64), :],
                            out_hbm.at[c, b, pl.ds(NS * WROWS, 64), :])
        plsc.subcore_barrier()


@jax.jit
def kernel(x, geom):
    x2 = x.reshape(NP, C)
    geomt = geom.reshape(NP, 3).T
    out = _bev_scatter(x2, geomt)
    out = out.reshape(NCORE, B, NX, NX, CH)
    return out.transpose(1, 0, 4, 2, 3).reshape(B, C, NX, NX)
